# baseline (speedup 1.0000x reference)
import functools

import jax
import jax.numpy as jnp
from jax.experimental import pallas as pl
from jax.experimental.pallas import tpu as pltpu
from jax.experimental.pallas import tpu_sc as plsc

BATCH = 4096
TILE = 4096
FEAT = 1024
EMB = 64
EMB_PAD = 128
N_TILES = BATCH // TILE


def _build_sc_gather():
    info = plsc.get_sparse_core_info()
    nc, ns = info.num_cores, info.num_subcores
    nw = nc * ns
    b_per_w = BATCH // nw

    mesh = plsc.VectorSubcoreMesh(core_axis_name="c", subcore_axis_name="s")

    @functools.partial(
        pl.kernel,
        mesh=mesh,
        out_type=jax.ShapeDtypeStruct((BATCH, EMB_PAD), jnp.float32),
        scratch_types=[
            pltpu.VMEM((b_per_w,), jnp.int32),
            pltpu.VMEM((b_per_w, EMB_PAD), jnp.float32),
            pltpu.SemaphoreType.DMA,
        ],
    )
    def gather_k(embed_hbm, idx_hbm, out_hbm, idx_v, rows_v, sem):
        wid = jax.lax.axis_index("s") * nc + jax.lax.axis_index("c")
        base = wid * b_per_w
        pltpu.sync_copy(idx_hbm.at[pl.ds(base, b_per_w)], idx_v)
        pltpu.async_copy(embed_hbm.at[idx_v], rows_v, sem).wait()
        pltpu.sync_copy(rows_v, out_hbm.at[pl.ds(base, b_per_w)])

    return gather_k


def _mlp_kernel(x0_ref, x1_ref, x2_ref, x3_ref, ye_ref, w0x_ref, w0e_ref,
                b0_ref, w1_ref, b1_ref, w2_ref, b2_ref, w3_ref, b3_ref,
                out_ref):
    h = jnp.dot(x0_ref[...], w0x_ref[0:256, :],
                preferred_element_type=jnp.float32)
    h = h + jnp.dot(x1_ref[...], w0x_ref[256:512, :],
                    preferred_element_type=jnp.float32)
    h = h + jnp.dot(x2_ref[...], w0x_ref[512:768, :],
                    preferred_element_type=jnp.float32)
    h = h + jnp.dot(x3_ref[...], w0x_ref[768:1024, :],
                    preferred_element_type=jnp.float32)
    h = h + jnp.dot(ye_ref[...], w0e_ref[...],
                    preferred_element_type=jnp.float32)
    h = h + b0_ref[...]
    h = jnp.maximum(h, 0.2 * h)
    h = jnp.dot(h, w1_ref[...], preferred_element_type=jnp.float32) + b1_ref[...]
    h = jnp.maximum(h, 0.2 * h)
    h = jnp.dot(h, w2_ref[...], preferred_element_type=jnp.float32) + b2_ref[...]
    h = jnp.maximum(h, 0.2 * h)
    o = jnp.dot(h, w3_ref[...], preferred_element_type=jnp.float32)
    out_ref[...] = o + b3_ref[...]


@jax.jit
def kernel(x, y, embed, W0, b0, W1, b1, W2, b2, W3, b3):
    embed_p = jnp.pad(embed, ((0, 0), (0, EMB_PAD - EMB)))
    w0e_p = jnp.pad(W0[FEAT:], ((0, EMB_PAD - EMB), (0, 0)))
    y_e = _build_sc_gather()(embed_p, y.astype(jnp.int32))
    out = pl.pallas_call(
        _mlp_kernel,
        grid=(N_TILES,),
        in_specs=[
            pl.BlockSpec((TILE, 256), lambda i: (i, 0)),
            pl.BlockSpec((TILE, 256), lambda i: (i, 1)),
            pl.BlockSpec((TILE, 256), lambda i: (i, 2)),
            pl.BlockSpec((TILE, 256), lambda i: (i, 3)),
            pl.BlockSpec((TILE, EMB_PAD), lambda i: (i, 0)),
            pl.BlockSpec((FEAT, 1024), lambda i: (0, 0)),
            pl.BlockSpec((EMB_PAD, 1024), lambda i: (0, 0)),
            pl.BlockSpec((1, 1024), lambda i: (0, 0)),
            pl.BlockSpec((1024, 512), lambda i: (0, 0)),
            pl.BlockSpec((1, 512), lambda i: (0, 0)),
            pl.BlockSpec((512, 256), lambda i: (0, 0)),
            pl.BlockSpec((1, 256), lambda i: (0, 0)),
            pl.BlockSpec((256, 1), lambda i: (0, 0)),
            pl.BlockSpec((1, 1), lambda i: (0, 0)),
        ],
        out_specs=pl.BlockSpec((TILE, 1), lambda i: (i, 0)),
        out_shape=jax.ShapeDtypeStruct((BATCH, 1), jnp.float32),
    )(x, x, x, x, y_e, W0[:FEAT], w0e_p, b0.reshape(1, -1), W1, b1.reshape(1, -1),
      W2, b2.reshape(1, -1), W3, b3.reshape(1, 1))
    return out.reshape(BATCH)


# speedup vs baseline: 1.0438x; 1.0438x over previous
import functools

import jax
import jax.numpy as jnp
from jax.experimental import pallas as pl
from jax.experimental.pallas import tpu as pltpu
from jax.experimental.pallas import tpu_sc as plsc

BATCH = 4096
TILE = 1024
FEAT = 1024
EMB = 64
EMB_PAD = 128
N_TILES = BATCH // TILE


def _build_sc_gather():
    info = plsc.get_sparse_core_info()
    nc, ns = info.num_cores, info.num_subcores
    nw = nc * ns
    b_per_w = BATCH // nw

    mesh = plsc.VectorSubcoreMesh(core_axis_name="c", subcore_axis_name="s")

    @functools.partial(
        pl.kernel,
        mesh=mesh,
        out_type=jax.ShapeDtypeStruct((BATCH, EMB_PAD), jnp.float32),
        scratch_types=[
            pltpu.VMEM((b_per_w,), jnp.int32),
            pltpu.VMEM((b_per_w, EMB_PAD), jnp.float32),
            pltpu.SemaphoreType.DMA,
        ],
    )
    def gather_k(embed_hbm, idx_hbm, out_hbm, idx_v, rows_v, sem):
        wid = jax.lax.axis_index("s") * nc + jax.lax.axis_index("c")
        base = wid * b_per_w
        pltpu.sync_copy(idx_hbm.at[pl.ds(base, b_per_w)], idx_v)
        pltpu.async_copy(embed_hbm.at[idx_v], rows_v, sem).wait()
        pltpu.sync_copy(rows_v, out_hbm.at[pl.ds(base, b_per_w)])

    return gather_k


def _mlp_kernel(x0_ref, x1_ref, x2_ref, x3_ref, ye_ref, w0x_ref, w0e_ref,
                b0_ref, w1_ref, b1_ref, w2_ref, b2_ref, w3_ref, b3_ref,
                out_ref):
    h = jnp.dot(x0_ref[...], w0x_ref[0:256, :],
                preferred_element_type=jnp.float32)
    h = h + jnp.dot(x1_ref[...], w0x_ref[256:512, :],
                    preferred_element_type=jnp.float32)
    h = h + jnp.dot(x2_ref[...], w0x_ref[512:768, :],
                    preferred_element_type=jnp.float32)
    h = h + jnp.dot(x3_ref[...], w0x_ref[768:1024, :],
                    preferred_element_type=jnp.float32)
    h = h + jnp.dot(ye_ref[...], w0e_ref[...],
                    preferred_element_type=jnp.float32)
    h = h + b0_ref[...]
    h = jnp.maximum(h, 0.2 * h)
    h = jnp.dot(h, w1_ref[...], preferred_element_type=jnp.float32) + b1_ref[...]
    h = jnp.maximum(h, 0.2 * h)
    h = jnp.dot(h, w2_ref[...], preferred_element_type=jnp.float32) + b2_ref[...]
    h = jnp.maximum(h, 0.2 * h)
    o = jnp.dot(h, w3_ref[...], preferred_element_type=jnp.float32)
    out_ref[...] = o + b3_ref[...]


@jax.jit
def kernel(x, y, embed, W0, b0, W1, b1, W2, b2, W3, b3):
    embed_p = jnp.pad(embed, ((0, 0), (0, EMB_PAD - EMB)))
    w0e_p = jnp.pad(W0[FEAT:], ((0, EMB_PAD - EMB), (0, 0)))
    y_e = _build_sc_gather()(embed_p, y.astype(jnp.int32))
    out = pl.pallas_call(
        _mlp_kernel,
        grid=(N_TILES,),
        in_specs=[
            pl.BlockSpec((TILE, 256), lambda i: (i, 0)),
            pl.BlockSpec((TILE, 256), lambda i: (i, 1)),
            pl.BlockSpec((TILE, 256), lambda i: (i, 2)),
            pl.BlockSpec((TILE, 256), lambda i: (i, 3)),
            pl.BlockSpec((TILE, EMB_PAD), lambda i: (i, 0)),
            pl.BlockSpec((FEAT, 1024), lambda i: (0, 0)),
            pl.BlockSpec((EMB_PAD, 1024), lambda i: (0, 0)),
            pl.BlockSpec((1, 1024), lambda i: (0, 0)),
            pl.BlockSpec((1024, 512), lambda i: (0, 0)),
            pl.BlockSpec((1, 512), lambda i: (0, 0)),
            pl.BlockSpec((512, 256), lambda i: (0, 0)),
            pl.BlockSpec((1, 256), lambda i: (0, 0)),
            pl.BlockSpec((256, 1), lambda i: (0, 0)),
            pl.BlockSpec((1, 1), lambda i: (0, 0)),
        ],
        out_specs=pl.BlockSpec((TILE, 1), lambda i: (i, 0)),
        out_shape=jax.ShapeDtypeStruct((BATCH, 1), jnp.float32),
        compiler_params=pltpu.CompilerParams(
            dimension_semantics=("parallel",)),
    )(x, x, x, x, y_e, W0[:FEAT], w0e_p, b0.reshape(1, -1), W1, b1.reshape(1, -1),
      W2, b2.reshape(1, -1), W3, b3.reshape(1, 1))
    return out.reshape(BATCH)


# speedup vs baseline: 1.1072x; 1.0607x over previous
import functools

import jax
import jax.numpy as jnp
from jax.experimental import pallas as pl
from jax.experimental.pallas import tpu as pltpu
from jax.experimental.pallas import tpu_sc as plsc

BATCH = 4096
TILE = 1024
FEAT = 1024
EMB = 64
EMB_PAD = 128
N_TILES = BATCH // TILE


def _build_sc_gather():
    info = plsc.get_sparse_core_info()
    nc, ns = info.num_cores, info.num_subcores
    nw = nc * ns
    b_per_w = BATCH // nw

    mesh = plsc.VectorSubcoreMesh(core_axis_name="c", subcore_axis_name="s")

    @functools.partial(
        pl.kernel,
        mesh=mesh,
        out_type=jax.ShapeDtypeStruct((BATCH, EMB_PAD), jnp.float32),
        scratch_types=[
            pltpu.VMEM((b_per_w,), jnp.int32),
            pltpu.VMEM((b_per_w, EMB_PAD), jnp.float32),
            pltpu.SemaphoreType.DMA,
        ],
    )
    def gather_k(embed_hbm, idx_hbm, out_hbm, idx_v, rows_v, sem):
        wid = jax.lax.axis_index("s") * nc + jax.lax.axis_index("c")
        base = wid * b_per_w
        pltpu.sync_copy(idx_hbm.at[pl.ds(base, b_per_w)], idx_v)
        pltpu.async_copy(embed_hbm.at[idx_v], rows_v, sem).wait()
        pltpu.sync_copy(rows_v, out_hbm.at[pl.ds(base, b_per_w)])

    return gather_k


def _mlp_kernel(x_ref, ye_ref, w0_ref, b0_ref, w1_ref,
                b1_ref, w2_ref, b2_ref, w3_ref, b3_ref, out_ref):
    h = jnp.dot(x_ref[...], w0_ref[0:FEAT, :],
                preferred_element_type=jnp.float32)
    h = h + jnp.dot(ye_ref[...], w0_ref[FEAT + EMB - EMB_PAD:FEAT + EMB, :],
                    preferred_element_type=jnp.float32)
    h = h + b0_ref[...]
    h = jnp.maximum(h, 0.2 * h)
    h = jnp.dot(h, w1_ref[...], preferred_element_type=jnp.float32) + b1_ref[...]
    h = jnp.maximum(h, 0.2 * h)
    h = jnp.dot(h, w2_ref[...], preferred_element_type=jnp.float32) + b2_ref[...]
    h = jnp.maximum(h, 0.2 * h)
    o = jnp.dot(h, w3_ref[...], preferred_element_type=jnp.float32)
    out_ref[...] = o + b3_ref[...]


@jax.jit
def kernel(x, y, embed, W0, b0, W1, b1, W2, b2, W3, b3):
    embed_p = jnp.pad(embed, ((0, 0), (EMB_PAD - EMB, 0)))
    y_e = _build_sc_gather()(embed_p, y.astype(jnp.int32))
    out = pl.pallas_call(
        _mlp_kernel,
        grid=(N_TILES,),
        in_specs=[
            pl.BlockSpec((TILE, FEAT), lambda i: (i, 0)),
            pl.BlockSpec((TILE, EMB_PAD), lambda i: (i, 0)),
            pl.BlockSpec((FEAT + EMB, 1024), lambda i: (0, 0)),
            pl.BlockSpec((1, 1024), lambda i: (0, 0)),
            pl.BlockSpec((1024, 512), lambda i: (0, 0)),
            pl.BlockSpec((1, 512), lambda i: (0, 0)),
            pl.BlockSpec((512, 256), lambda i: (0, 0)),
            pl.BlockSpec((1, 256), lambda i: (0, 0)),
            pl.BlockSpec((256, 1), lambda i: (0, 0)),
            pl.BlockSpec((1, 1), lambda i: (0, 0)),
        ],
        out_specs=pl.BlockSpec((TILE, 1), lambda i: (i, 0)),
        out_shape=jax.ShapeDtypeStruct((BATCH, 1), jnp.float32),
        compiler_params=pltpu.CompilerParams(
            dimension_semantics=("parallel",)),
    )(x, y_e, W0, b0.reshape(1, -1), W1, b1.reshape(1, -1),
      W2, b2.reshape(1, -1), W3, b3.reshape(1, 1))
    return out.reshape(BATCH)


# speedup vs baseline: 1.1689x; 1.0557x over previous
import functools

import jax
import jax.numpy as jnp
from jax.experimental import pallas as pl
from jax.experimental.pallas import tpu as pltpu
from jax.experimental.pallas import tpu_sc as plsc

BATCH = 4096
TILE = 1024
FEAT = 1024
EMB = 64
EMB_PAD = 128
N_TILES = BATCH // TILE


def _build_sc_gather():
    info = plsc.get_sparse_core_info()
    nc, ns = info.num_cores, info.num_subcores
    nw = nc * ns
    b_per_w = BATCH // nw

    mesh = plsc.VectorSubcoreMesh(core_axis_name="c", subcore_axis_name="s")

    @functools.partial(
        pl.kernel,
        mesh=mesh,
        out_type=jax.ShapeDtypeStruct((BATCH, EMB_PAD), jnp.float32),
        scratch_types=[
            pltpu.VMEM((b_per_w,), jnp.int32),
            pltpu.VMEM((b_per_w, EMB_PAD), jnp.float32),
            pltpu.SemaphoreType.DMA,
        ],
    )
    def gather_k(embed_hbm, idx_hbm, out_hbm, idx_v, rows_v, sem):
        wid = jax.lax.axis_index("s") * nc + jax.lax.axis_index("c")
        base = wid * b_per_w
        pltpu.sync_copy(idx_hbm.at[pl.ds(base, b_per_w)], idx_v)
        pltpu.async_copy(embed_hbm.at[idx_v], rows_v, sem).wait()
        pltpu.sync_copy(rows_v, out_hbm.at[pl.ds(base, b_per_w)])

    return gather_k


def _mlp_kernel(x_ref, ye_ref, w0_ref, b0_ref, w1_ref,
                b1_ref, w2_ref, b2_ref, w3_ref, b3_ref, out_ref):
    h = jnp.dot(x_ref[...], w0_ref[0:FEAT, :],
                preferred_element_type=jnp.float32)
    h = h + jnp.dot(ye_ref[...], w0_ref[FEAT + EMB - EMB_PAD:FEAT + EMB, :],
                    preferred_element_type=jnp.float32)
    h = h + b0_ref[...]
    h = jnp.maximum(h, 0.2 * h)
    h = jnp.dot(h, w1_ref[...], preferred_element_type=jnp.float32) + b1_ref[...]
    h = jnp.maximum(h, 0.2 * h)
    h = jnp.dot(h, w2_ref[...], preferred_element_type=jnp.float32) + b2_ref[...]
    h = jnp.maximum(h, 0.2 * h)
    o = jnp.dot(h, w3_ref[...], preferred_element_type=jnp.float32)
    out_ref[...] = (o + b3_ref[...]).reshape(1, TILE)


@jax.jit
def kernel(x, y, embed, W0, b0, W1, b1, W2, b2, W3, b3):
    embed_p = jnp.pad(embed, ((0, 0), (EMB_PAD - EMB, 0)))
    y_e = _build_sc_gather()(embed_p, y.astype(jnp.int32))
    out = pl.pallas_call(
        _mlp_kernel,
        grid=(N_TILES,),
        in_specs=[
            pl.BlockSpec((TILE, FEAT), lambda i: (i, 0)),
            pl.BlockSpec((TILE, EMB_PAD), lambda i: (i, 0)),
            pl.BlockSpec((FEAT + EMB, 1024), lambda i: (0, 0)),
            pl.BlockSpec((1, 1024), lambda i: (0, 0)),
            pl.BlockSpec((1024, 512), lambda i: (0, 0)),
            pl.BlockSpec((1, 512), lambda i: (0, 0)),
            pl.BlockSpec((512, 256), lambda i: (0, 0)),
            pl.BlockSpec((1, 256), lambda i: (0, 0)),
            pl.BlockSpec((256, 1), lambda i: (0, 0)),
            pl.BlockSpec((1, 1), lambda i: (0, 0)),
        ],
        out_specs=pl.BlockSpec((1, TILE), lambda i: (0, i)),
        out_shape=jax.ShapeDtypeStruct((1, BATCH), jnp.float32),
        compiler_params=pltpu.CompilerParams(
            dimension_semantics=("parallel",)),
    )(x, y_e, W0, b0.reshape(1, -1), W1, b1.reshape(1, -1),
      W2, b2.reshape(1, -1), W3, b3.reshape(1, 1))
    return out.reshape(BATCH)
